# baseline (speedup 1.0000x reference)
import functools

import jax
import jax.numpy as jnp
from jax import lax
from jax.experimental import pallas as pl
from jax.experimental.pallas import tpu as pltpu
from jax.experimental.pallas import tpu_sc as plsc

D = 128
NC = 2
NS = 16
NW = NC * NS
CH = 128


def _sc_gather(table, ids_t):
  n_ch, b = ids_t.shape
  total = n_ch * b
  mesh = plsc.VectorSubcoreMesh(
      core_axis_name="c", subcore_axis_name="s", num_cores=NC, num_subcores=NS
  )

  @functools.partial(
      pl.kernel,
      out_type=jax.ShapeDtypeStruct((total, D), jnp.float32),
      mesh=mesh,
      scratch_types=[
          pltpu.VMEM((n_ch, CH), jnp.int32),
          pltpu.VMEM((CH, D), jnp.float32),
          pltpu.VMEM((CH, D), jnp.float32),
          pltpu.SemaphoreType.DMA,
          pltpu.SemaphoreType.DMA,
      ],
  )
  def k(table_hbm, ids_hbm, out_hbm, idx_v, rows0, rows1, sem0, sem1):
    wid = lax.axis_index("s") * NC + lax.axis_index("c")
    col0 = wid * CH
    pltpu.sync_copy(ids_hbm.at[pl.ds(0, n_ch), pl.ds(col0, CH)], idx_v)
    pltpu.async_copy(table_hbm.at[idx_v.at[0]], rows0, sem0)
    pltpu.async_copy(table_hbm.at[idx_v.at[1]], rows1, sem1)

    def pair(p, _):
      g0 = 2 * p

      pltpu.make_async_copy(table_hbm.at[idx_v.at[g0]], rows0, sem0).wait()
      pltpu.sync_copy(rows0, out_hbm.at[pl.ds(g0 * b + col0, CH)])

      @pl.when(g0 + 2 < n_ch)
      def _():
        pltpu.async_copy(table_hbm.at[idx_v.at[g0 + 2]], rows0, sem0)

      pltpu.make_async_copy(
          table_hbm.at[idx_v.at[g0 + 1]], rows1, sem1).wait()
      pltpu.sync_copy(rows1, out_hbm.at[pl.ds((g0 + 1) * b + col0, CH)])

      @pl.when(g0 + 3 < n_ch)
      def _():
        pltpu.async_copy(table_hbm.at[idx_v.at[g0 + 3]], rows1, sem1)

      return ()

    lax.fori_loop(0, n_ch // 2, pair, ())

    if n_ch % 2:
      g_last = n_ch - 1
      pltpu.make_async_copy(
          table_hbm.at[idx_v.at[g_last]], rows0, sem0).wait()
      pltpu.sync_copy(rows0, out_hbm.at[pl.ds(g_last * b + col0, CH)])

  return k(table, ids_t)


def _tc_main(x, g, wa, ba, wt, bt, rows_per_tile, tile0, n_tiles, l_seg):
  r = rows_per_tile
  b = (n_tiles * r) // l_seg
  tpl = b // r

  def body(x_ref, g_ref, wa_ref, ba_ref, wt_ref, bt_ref, out_ref, m_s):
    @pl.when(pl.program_id(0) == 0)
    def _():
      m_s[...] = lax.dot_general(
          wa_ref[...], wt_ref[...], (((0,), (0,)), ((), ())),
          preferred_element_type=jnp.float32)

    xv = x_ref[...]
    gv = g_ref[...]
    v = jnp.dot(ba_ref[...], wt_ref[...], preferred_element_type=jnp.float32)
    u = jnp.dot(bt_ref[...], wa_ref[...], preferred_element_type=jnp.float32)
    c = jnp.sum(ba_ref[...] * bt_ref[...])
    a = jnp.dot(xv, m_s[...], preferred_element_type=jnp.float32) + v
    ones = jnp.ones((1, D), dtype=jnp.float32)
    res = lax.dot_general(
        ones, a * gv, (((1,), (1,)), ((), ())),
        preferred_element_type=jnp.float32)
    z = lax.dot_general(
        u, xv, (((1,), (1,)), ((), ())),
        preferred_element_type=jnp.float32)
    out_ref[...] = (res + z + c).reshape(1, 1, r)

  out = pl.pallas_call(
      body,
      grid=(n_tiles,),
      in_specs=[
          pl.BlockSpec((r, D), lambda i: (tile0 + i, 0)),
          pl.BlockSpec((r, D), lambda i: (i, 0)),
          pl.BlockSpec((D, D), lambda i: (0, 0)),
          pl.BlockSpec((1, D), lambda i: (0, 0)),
          pl.BlockSpec((D, D), lambda i: (0, 0)),
          pl.BlockSpec((1, D), lambda i: (0, 0)),
      ],
      out_specs=pl.BlockSpec((1, 1, r), lambda i: (i // tpl, 0, i % tpl)),
      out_shape=jax.ShapeDtypeStruct((l_seg, 1, b), jnp.float32),
      scratch_shapes=[pltpu.VMEM((D, D), jnp.float32)],
  )(x, g, wa, ba, wt, bt)
  return out


def kernel(actor_emb, topic_ids, Wa, ba, table, Wt, bt, scale):
  b, l, d = actor_emb.shape
  bl = b * l

  wa_s = Wa * scale
  ba_s = (ba * scale).reshape(1, d)

  ids_t = topic_ids.T.astype(jnp.int32)
  x = actor_emb.transpose(1, 0, 2).reshape(bl, d)

  n_seg = 10
  l_seg = l // n_seg
  r = 2048
  nt_seg = l_seg * b // r
  bt_r = bt.reshape(1, d)
  outs = []
  for s in range(n_seg):
    ids_seg = lax.slice_in_dim(ids_t, s * l_seg, (s + 1) * l_seg, axis=0)
    g_seg = _sc_gather(table, ids_seg)
    outs.append(_tc_main(x, g_seg, wa_s, ba_s, Wt, bt_r, r,
                         s * nt_seg, nt_seg, l_seg))
  out = jnp.concatenate(outs, axis=0)
  return out.reshape(l, b).T


# speedup vs baseline: 1.0130x; 1.0130x over previous
import functools

import jax
import jax.numpy as jnp
from jax import lax
from jax.experimental import pallas as pl
from jax.experimental.pallas import tpu as pltpu
from jax.experimental.pallas import tpu_sc as plsc

D = 128
NC = 2
NS = 16
NW = NC * NS
CH = 128
NBUF = 4


def _sc_gather(table, ids_t):
  n_ch, b = ids_t.shape
  total = n_ch * b
  mesh = plsc.VectorSubcoreMesh(
      core_axis_name="c", subcore_axis_name="s", num_cores=NC, num_subcores=NS
  )

  @functools.partial(
      pl.kernel,
      out_type=jax.ShapeDtypeStruct((total, D), jnp.float32),
      mesh=mesh,
      scratch_types=[
          pltpu.VMEM((n_ch, CH), jnp.int32),
          [pltpu.VMEM((CH, D), jnp.float32) for _ in range(NBUF)],
          [pltpu.SemaphoreType.DMA for _ in range(NBUF)],
      ],
  )
  def k(table_hbm, ids_hbm, out_hbm, idx_v, rows, sems):
    wid = lax.axis_index("s") * NC + lax.axis_index("c")
    col0 = wid * CH
    pltpu.sync_copy(ids_hbm.at[pl.ds(0, n_ch), pl.ds(col0, CH)], idx_v)
    for j in range(min(NBUF, n_ch)):
      pltpu.async_copy(table_hbm.at[idx_v.at[j]], rows[j], sems[j])

    def quad(q, _):
      j0 = NBUF * q
      for t in range(NBUF):
        j = j0 + t
        pltpu.make_async_copy(
            table_hbm.at[idx_v.at[j]], rows[t], sems[t]).wait()
        pltpu.sync_copy(rows[t], out_hbm.at[pl.ds(j * b + col0, CH)])

        @pl.when(j + NBUF < n_ch)
        def _():
          pltpu.async_copy(table_hbm.at[idx_v.at[j + NBUF]], rows[t], sems[t])

      return ()

    lax.fori_loop(0, n_ch // NBUF, quad, ())

    for t in range(n_ch % NBUF):
      j = (n_ch // NBUF) * NBUF + t
      pltpu.make_async_copy(
          table_hbm.at[idx_v.at[j]], rows[t], sems[t]).wait()
      pltpu.sync_copy(rows[t], out_hbm.at[pl.ds(j * b + col0, CH)])

  return k(table, ids_t)


def _tc_main(x, g, wa, ba, wt, bt, rows_per_tile, tile0, n_tiles, l_seg):
  r = rows_per_tile
  b = (n_tiles * r) // l_seg
  tpl = b // r

  def body(x_ref, g_ref, wa_ref, ba_ref, wt_ref, bt_ref, out_ref, m_s):
    @pl.when(pl.program_id(0) == 0)
    def _():
      m_s[...] = lax.dot_general(
          wa_ref[...], wt_ref[...], (((0,), (0,)), ((), ())),
          preferred_element_type=jnp.float32)

    xv = x_ref[...]
    gv = g_ref[...]
    v = jnp.dot(ba_ref[...], wt_ref[...], preferred_element_type=jnp.float32)
    u = jnp.dot(bt_ref[...], wa_ref[...], preferred_element_type=jnp.float32)
    c = jnp.sum(ba_ref[...] * bt_ref[...])
    a = jnp.dot(xv, m_s[...], preferred_element_type=jnp.float32) + v
    ones = jnp.ones((1, D), dtype=jnp.float32)
    res = lax.dot_general(
        ones, a * gv, (((1,), (1,)), ((), ())),
        preferred_element_type=jnp.float32)
    z = lax.dot_general(
        u, xv, (((1,), (1,)), ((), ())),
        preferred_element_type=jnp.float32)
    out_ref[...] = (res + z + c).reshape(1, 1, r)

  out = pl.pallas_call(
      body,
      grid=(n_tiles,),
      in_specs=[
          pl.BlockSpec((r, D), lambda i: (tile0 + i, 0)),
          pl.BlockSpec((r, D), lambda i: (i, 0)),
          pl.BlockSpec((D, D), lambda i: (0, 0)),
          pl.BlockSpec((1, D), lambda i: (0, 0)),
          pl.BlockSpec((D, D), lambda i: (0, 0)),
          pl.BlockSpec((1, D), lambda i: (0, 0)),
      ],
      out_specs=pl.BlockSpec((1, 1, r), lambda i: (i // tpl, 0, i % tpl)),
      out_shape=jax.ShapeDtypeStruct((l_seg, 1, b), jnp.float32),
      scratch_shapes=[pltpu.VMEM((D, D), jnp.float32)],
  )(x, g, wa, ba, wt, bt)
  return out


def kernel(actor_emb, topic_ids, Wa, ba, table, Wt, bt, scale):
  b, l, d = actor_emb.shape
  bl = b * l

  wa_s = Wa * scale
  ba_s = (ba * scale).reshape(1, d)

  ids_t = topic_ids.T.astype(jnp.int32)
  x = actor_emb.transpose(1, 0, 2).reshape(bl, d)

  n_seg = 5
  l_seg = l // n_seg
  r = 2048
  nt_seg = l_seg * b // r
  bt_r = bt.reshape(1, d)
  outs = []
  for s in range(n_seg):
    ids_seg = lax.slice_in_dim(ids_t, s * l_seg, (s + 1) * l_seg, axis=0)
    g_seg = _sc_gather(table, ids_seg)
    outs.append(_tc_main(x, g_seg, wa_s, ba_s, Wt, bt_r, r,
                         s * nt_seg, nt_seg, l_seg))
  out = jnp.concatenate(outs, axis=0)
  return out.reshape(l, b).T


# speedup vs baseline: 1.1430x; 1.1283x over previous
import functools

import jax
import jax.numpy as jnp
from jax import lax
from jax.experimental import pallas as pl
from jax.experimental.pallas import tpu as pltpu
from jax.experimental.pallas import tpu_sc as plsc

D = 128
NC = 2
NS = 16
NW = NC * NS
CH = 128
NBUF = 4


def _sc_gather(table, ids_t):
  n_ch, b = ids_t.shape
  total = n_ch * b
  mesh = plsc.VectorSubcoreMesh(
      core_axis_name="c", subcore_axis_name="s", num_cores=NC, num_subcores=NS
  )

  @functools.partial(
      pl.kernel,
      out_type=jax.ShapeDtypeStruct((total, D), jnp.float32),
      mesh=mesh,
      scratch_types=[
          pltpu.VMEM((n_ch, CH), jnp.int32),
          [pltpu.VMEM((CH, D), jnp.float32) for _ in range(NBUF)],
          [pltpu.SemaphoreType.DMA for _ in range(NBUF)],
      ],
  )
  def k(table_hbm, ids_hbm, out_hbm, idx_v, rows, sems):
    wid = lax.axis_index("s") * NC + lax.axis_index("c")
    col0 = wid * CH
    pltpu.sync_copy(ids_hbm.at[pl.ds(0, n_ch), pl.ds(col0, CH)], idx_v)
    for j in range(min(NBUF, n_ch)):
      pltpu.async_copy(table_hbm.at[idx_v.at[j]], rows[j], sems[j])

    def quad(q, _):
      j0 = NBUF * q
      for t in range(NBUF):
        j = j0 + t
        pltpu.make_async_copy(
            table_hbm.at[idx_v.at[j]], rows[t], sems[t]).wait()
        pltpu.sync_copy(rows[t], out_hbm.at[pl.ds(j * b + col0, CH)])

        @pl.when(j + NBUF < n_ch)
        def _():
          pltpu.async_copy(table_hbm.at[idx_v.at[j + NBUF]], rows[t], sems[t])

      return ()

    lax.fori_loop(0, n_ch // NBUF, quad, ())

    for t in range(n_ch % NBUF):
      j = (n_ch // NBUF) * NBUF + t
      pltpu.make_async_copy(
          table_hbm.at[idx_v.at[j]], rows[t], sems[t]).wait()
      pltpu.sync_copy(rows[t], out_hbm.at[pl.ds(j * b + col0, CH)])

  return k(table, ids_t)


def _tc_main(x, g, wa, ba, wt, bt, rows_per_tile, tile0, n_tiles, l_seg):
  r = rows_per_tile
  b = (n_tiles * r) // l_seg
  tpl = b // r

  def body(x_ref, g_ref, wa_ref, ba_ref, wt_ref, bt_ref, out_ref, m_s):
    @pl.when(pl.program_id(0) == 0)
    def _():
      m_s[...] = lax.dot_general(
          wa_ref[...], wt_ref[...], (((0,), (0,)), ((), ())),
          preferred_element_type=jnp.float32)

    xv = x_ref[...]
    gv = g_ref[...]
    v = jnp.dot(ba_ref[...], wt_ref[...], preferred_element_type=jnp.float32)
    u = jnp.dot(bt_ref[...], wa_ref[...], preferred_element_type=jnp.float32)
    c = jnp.sum(ba_ref[...] * bt_ref[...])
    a = jnp.dot(xv, m_s[...], preferred_element_type=jnp.float32) + v
    ones = jnp.ones((1, D), dtype=jnp.float32)
    res = lax.dot_general(
        ones, a * gv, (((1,), (1,)), ((), ())),
        preferred_element_type=jnp.float32)
    z = lax.dot_general(
        u, xv, (((1,), (1,)), ((), ())),
        preferred_element_type=jnp.float32)
    out_ref[...] = (res + z + c).reshape(1, 1, r)

  out = pl.pallas_call(
      body,
      grid=(n_tiles,),
      in_specs=[
          pl.BlockSpec((r, D), lambda i: (tile0 + i, 0)),
          pl.BlockSpec((r, D), lambda i: (i, 0)),
          pl.BlockSpec((D, D), lambda i: (0, 0)),
          pl.BlockSpec((1, D), lambda i: (0, 0)),
          pl.BlockSpec((D, D), lambda i: (0, 0)),
          pl.BlockSpec((1, D), lambda i: (0, 0)),
      ],
      out_specs=pl.BlockSpec((1, 1, r), lambda i: (i // tpl, 0, i % tpl)),
      out_shape=jax.ShapeDtypeStruct((l_seg, 1, b), jnp.float32),
      scratch_shapes=[pltpu.VMEM((D, D), jnp.float32)],
  )(x, g, wa, ba, wt, bt)
  return out


def kernel(actor_emb, topic_ids, Wa, ba, table, Wt, bt, scale):
  b, l, d = actor_emb.shape
  bl = b * l

  wa_s = Wa * scale
  ba_s = (ba * scale).reshape(1, d)

  ids_t = topic_ids.T.astype(jnp.int32)
  x = actor_emb.transpose(1, 0, 2).reshape(bl, d)

  n_seg = 5
  l_seg = l // n_seg
  r = 4096
  nt_seg = l_seg * b // r
  bt_r = bt.reshape(1, d)
  outs = []
  for s in range(n_seg):
    ids_seg = lax.slice_in_dim(ids_t, s * l_seg, (s + 1) * l_seg, axis=0)
    g_seg = _sc_gather(table, ids_seg)
    outs.append(_tc_main(x, g_seg, wa_s, ba_s, Wt, bt_r, r,
                         s * nt_seg, nt_seg, l_seg))
  out = jnp.concatenate(outs, axis=0)
  return out.reshape(l, b).T


# speedup vs baseline: 1.1610x; 1.0157x over previous
import functools

import jax
import jax.numpy as jnp
from jax import lax
from jax.experimental import pallas as pl
from jax.experimental.pallas import tpu as pltpu
from jax.experimental.pallas import tpu_sc as plsc

D = 128
NC = 2
NS = 16
NW = NC * NS
CH = 128
NBUF = 4


def _sc_gather(table, ids_t):
  n_ch, b = ids_t.shape
  total = n_ch * b
  mesh = plsc.VectorSubcoreMesh(
      core_axis_name="c", subcore_axis_name="s", num_cores=NC, num_subcores=NS
  )

  @functools.partial(
      pl.kernel,
      out_type=jax.ShapeDtypeStruct((total, D), jnp.float32),
      mesh=mesh,
      scratch_types=[
          pltpu.VMEM((n_ch, CH), jnp.int32),
          [pltpu.VMEM((CH, D), jnp.float32) for _ in range(NBUF)],
          [pltpu.SemaphoreType.DMA for _ in range(NBUF)],
      ],
  )
  def k(table_hbm, ids_hbm, out_hbm, idx_v, rows, sems):
    wid = lax.axis_index("s") * NC + lax.axis_index("c")
    col0 = wid * CH
    pltpu.sync_copy(ids_hbm.at[pl.ds(0, n_ch), pl.ds(col0, CH)], idx_v)
    for j in range(min(NBUF, n_ch)):
      pltpu.async_copy(table_hbm.at[idx_v.at[j]], rows[j], sems[j])

    def quad(q, _):
      j0 = NBUF * q
      for t in range(NBUF):
        j = j0 + t
        pltpu.make_async_copy(
            table_hbm.at[idx_v.at[j]], rows[t], sems[t]).wait()
        pltpu.sync_copy(rows[t], out_hbm.at[pl.ds(j * b + col0, CH)])

        @pl.when(j + NBUF < n_ch)
        def _():
          pltpu.async_copy(table_hbm.at[idx_v.at[j + NBUF]], rows[t], sems[t])

      return ()

    lax.fori_loop(0, n_ch // NBUF, quad, ())

    for t in range(n_ch % NBUF):
      j = (n_ch // NBUF) * NBUF + t
      pltpu.make_async_copy(
          table_hbm.at[idx_v.at[j]], rows[t], sems[t]).wait()
      pltpu.sync_copy(rows[t], out_hbm.at[pl.ds(j * b + col0, CH)])

  return k(table, ids_t)


def _tc_main(x, g, wa, ba, wt, bt, rows_per_tile, tile0, n_tiles, l_seg):
  r = rows_per_tile
  b = (n_tiles * r) // l_seg

  def body(x_ref, g_ref, wa_ref, ba_ref, wt_ref, bt_ref, out_ref, m_s):
    @pl.when(pl.program_id(0) == 0)
    def _():
      m_s[...] = lax.dot_general(
          wa_ref[...], wt_ref[...], (((0,), (0,)), ((), ())),
          preferred_element_type=jnp.float32)

    xv = x_ref[...]
    gv = g_ref[...]
    v = jnp.dot(ba_ref[...], wt_ref[...], preferred_element_type=jnp.float32)
    u = jnp.dot(bt_ref[...], wa_ref[...], preferred_element_type=jnp.float32)
    c = jnp.sum(ba_ref[...] * bt_ref[...])
    a = jnp.dot(xv, m_s[...], preferred_element_type=jnp.float32) + v
    ones = jnp.ones((1, D), dtype=jnp.float32)
    res = lax.dot_general(
        ones, a * gv, (((1,), (1,)), ((), ())),
        preferred_element_type=jnp.float32)
    z = lax.dot_general(
        u, xv, (((1,), (1,)), ((), ())),
        preferred_element_type=jnp.float32)
    if r >= b:
      out_ref[...] = (res + z + c).reshape(r // b, 1, b)
    else:
      out_ref[...] = (res + z + c).reshape(1, 1, r)

  out = pl.pallas_call(
      body,
      grid=(n_tiles,),
      in_specs=[
          pl.BlockSpec((r, D), lambda i: (tile0 + i, 0)),
          pl.BlockSpec((r, D), lambda i: (i, 0)),
          pl.BlockSpec((D, D), lambda i: (0, 0)),
          pl.BlockSpec((1, D), lambda i: (0, 0)),
          pl.BlockSpec((D, D), lambda i: (0, 0)),
          pl.BlockSpec((1, D), lambda i: (0, 0)),
      ],
      out_specs=(
          pl.BlockSpec((r // b, 1, b), lambda i: (i, 0, 0))
          if r >= b else
          pl.BlockSpec((1, 1, r), lambda i: (i // (b // r), 0, i % (b // r)))
      ),
      out_shape=jax.ShapeDtypeStruct((l_seg, 1, b), jnp.float32),
      scratch_shapes=[pltpu.VMEM((D, D), jnp.float32)],
  )(x, g, wa, ba, wt, bt)
  return out


def kernel(actor_emb, topic_ids, Wa, ba, table, Wt, bt, scale):
  b, l, d = actor_emb.shape
  bl = b * l

  wa_s = Wa * scale
  ba_s = (ba * scale).reshape(1, d)

  ids_t = topic_ids.T.astype(jnp.int32)
  x = actor_emb.transpose(1, 0, 2).reshape(bl, d)

  n_seg = 5
  l_seg = l // n_seg
  r = 8192
  nt_seg = l_seg * b // r
  bt_r = bt.reshape(1, d)
  outs = []
  for s in range(n_seg):
    ids_seg = lax.slice_in_dim(ids_t, s * l_seg, (s + 1) * l_seg, axis=0)
    g_seg = _sc_gather(table, ids_seg)
    outs.append(_tc_main(x, g_seg, wa_s, ba_s, Wt, bt_r, r,
                         s * nt_seg, nt_seg, l_seg))
  out = jnp.concatenate(outs, axis=0)
  return out.reshape(l, b).T
